# baseline (speedup 1.0000x reference)
import functools

import numpy as np
import jax
import jax.numpy as jnp
from jax import lax
from jax.experimental import pallas as pl
from jax.experimental.pallas import tpu as pltpu
from jax.experimental.pallas import tpu_sc as plsc

H = 224
G = 14
N_SEG = G * G
MAX_PIX = 400
N_CH = 3
SEG_COLS = N_CH * MAX_PIX
BAND_W = 17 * H
SENTINEL = N_CH * BAND_W
BAND_BUF = SENTINEL + 16
UNIT_COLS = G * SEG_COLS
SEG_COLS_P = 1280
UNIT_COLS_P = G * SEG_COLS_P
B_TOTAL = 64
UNITS = B_TOTAL * G
FEAT_ROWS = B_TOTAL * N_SEG


def _band_info():
    ys = (np.arange(G) + 0.5) * H / G
    seg = np.argmin(np.abs(np.arange(H)[:, None].astype(np.float32) - ys[None, :]), axis=1)
    out = []
    for k in range(G):
        rows = np.where(seg == k)[0]
        assert np.all(np.diff(rows) == 1)
        out.append((int(rows[0]), len(rows)))
    return out


def _build_idx_tab():
    bands = _band_info()
    tab = np.full((G, UNIT_COLS), SENTINEL, dtype=np.int32)
    yload_tab = np.zeros((G,), dtype=np.int32)
    for by, (y0, h) in enumerate(bands):
        yload = min(y0, H - 17)
        yload_tab[by] = yload
        roff = y0 - yload
        for bx, (x0, w) in enumerate(bands):
            n = h * w
            j = np.arange(n)
            off = (roff + j // w) * H + (x0 + j % w)
            for c in range(N_CH):
                base = bx * SEG_COLS + c * MAX_PIX
                tab[by, base:base + n] = c * BAND_W + off
    return tab, yload_tab


_IDX_TAB, _YLOAD_TAB = _build_idx_tab()
assert np.all(_YLOAD_TAB == np.minimum(np.where(np.arange(G) > 0, np.arange(G) * 16 + 1, 0), H - 17))


def _sc_gather(imgf, idx_tab):
    info = plsc.get_sparse_core_info()
    nw = info.num_cores * info.num_subcores
    assert UNITS % nw == 0
    per = UNITS // nw
    mesh = plsc.VectorSubcoreMesh(core_axis_name="c", subcore_axis_name="s")

    @functools.partial(
        pl.kernel,
        mesh=mesh,
        compiler_params=pltpu.CompilerParams(needs_layout_passes=False),
        out_type=jax.ShapeDtypeStruct((FEAT_ROWS * SEG_COLS_P,), jnp.float32),
        scratch_types=[
            pltpu.VMEM((BAND_BUF,), jnp.float32),
            pltpu.VMEM((BAND_BUF,), jnp.float32),
            pltpu.VMEM((UNIT_COLS,), jnp.int32),
            pltpu.VMEM((UNIT_COLS,), jnp.int32),
            pltpu.VMEM((UNIT_COLS_P,), jnp.float32),
            pltpu.VMEM((UNIT_COLS_P,), jnp.float32),
            pltpu.SemaphoreType.DMA,
            pltpu.SemaphoreType.DMA,
            pltpu.SemaphoreType.DMA,
            pltpu.SemaphoreType.DMA,
        ],
    )
    def k(img_hbm, tab_hbm, out_hbm, band0_v, band1_v, idx0_v, idx1_v,
          buf0_v, buf1_v, sb0, sb1, so0, so1):
        wid = lax.axis_index("s") * info.num_cores + lax.axis_index("c")
        u0 = wid * per
        bands_v = (band0_v, band1_v)
        idxs_v = (idx0_v, idx1_v)
        bufs_v = (buf0_v, buf1_v)
        sbands = (sb0, sb1)
        souts = (so0, so1)
        for p in range(2):
            bands_v[p][pl.ds(SENTINEL, 16)] = jnp.zeros((16,), jnp.float32)

        def unit_scalars(i):
            u = u0 + i
            by = u // B_TOTAL
            b = u - by * B_TOTAL
            y0 = jnp.where(by > 0, by * 16 + 1, 0)
            yload = jnp.minimum(y0, H - 17)
            return u, by, b, yload

        def start_band(i, p):
            _, by, b, yload = unit_scalars(i)
            copies = []
            for c in range(N_CH):
                copies.append(pltpu.async_copy(
                    img_hbm.at[pl.ds((b * N_CH + c) * (H * H) + yload * H, BAND_W)],
                    bands_v[p].at[pl.ds(c * BAND_W, BAND_W)],
                    sbands[p],
                ))
            copies.append(pltpu.async_copy(
                tab_hbm.at[pl.ds(by * UNIT_COLS, UNIT_COLS)], idxs_v[p], sbands[p]))
            return copies

        pend_band = {0: start_band(0, 0)}
        pend_out = {}

        for i in range(per):
            p = i & 1
            u, by, b, yload = unit_scalars(i)
            for h in pend_band.pop(i):
                h.wait()
            if i + 1 < per:
                pend_band[i + 1] = start_band(i + 1, 1 - p)
            if i - 2 in pend_out:
                pend_out.pop(i - 2).wait()

            def seg_body(bx, _1):
                def gather_body(kk, _2):
                    src = bx * SEG_COLS + kk * 48
                    dst = bx * SEG_COLS_P + kk * 48
                    for t in range(3):
                        ind = idxs_v[p][pl.ds(src + t * 16, 16)]
                        bufs_v[p][pl.ds(dst + t * 16, 16)] = plsc.load_gather(
                            bands_v[p], [ind])
                    return 0

                lax.fori_loop(0, SEG_COLS // 48, gather_body, 0, unroll=False)
                return 0

            lax.fori_loop(0, G, seg_body, 0, unroll=False)
            pend_out[i] = pltpu.async_copy(
                bufs_v[p],
                out_hbm.at[pl.ds((b * N_SEG + by * G) * SEG_COLS_P, UNIT_COLS_P)],
                souts[p])

        for h in pend_out.values():
            h.wait()

    return k(imgf, idx_tab)


def _tc_matmul(feats2d, Wp, bias2):
    BB = 4
    RB = BB * N_SEG

    def body(f_ref, w_ref, b_ref, o_ref):
        acc = lax.dot_general(
            f_ref[...], w_ref[...],
            (((1,), (0,)), ((), ())),
            preferred_element_type=jnp.float32,
        ) + b_ref[...]
        o_ref[...] = acc.reshape(BB, N_SEG, 128)

    return pl.pallas_call(
        body,
        grid=(B_TOTAL // BB,),
        in_specs=[
            pl.BlockSpec((RB, SEG_COLS_P), lambda i: (i, 0)),
            pl.BlockSpec((SEG_COLS_P, 128), lambda i: (0, 0)),
            pl.BlockSpec((1, 128), lambda i: (0, 0)),
        ],
        out_specs=pl.BlockSpec((BB, N_SEG, 128), lambda i: (i, 0, 0)),
        out_shape=jax.ShapeDtypeStruct((B_TOTAL, N_SEG, 128), jnp.float32),
    )(feats2d, Wp, bias2)


def kernel(img, W, b):
    imgf = img.reshape(B_TOTAL * N_CH * H * H)
    featsf = _sc_gather(imgf, jnp.asarray(_IDX_TAB).reshape(-1))
    feats2d = featsf.reshape(FEAT_ROWS, SEG_COLS_P)
    Wp = jnp.concatenate(
        [W, jnp.zeros((SEG_COLS_P - SEG_COLS, 128), W.dtype)], axis=0)
    return _tc_matmul(feats2d, Wp, b.reshape(1, 128))


# speedup vs baseline: 1.3481x; 1.3481x over previous
import functools

import numpy as np
import jax
import jax.numpy as jnp
from jax import lax
from jax.experimental import pallas as pl
from jax.experimental.pallas import tpu as pltpu
from jax.experimental.pallas import tpu_sc as plsc

H = 224
G = 14
N_SEG = G * G
MAX_PIX = 400
N_CH = 3
SEG_COLS = N_CH * MAX_PIX
BAND_W = 17 * H
SENTINEL = N_CH * BAND_W
BAND_BUF = SENTINEL + 16
UNIT_COLS = G * SEG_COLS
B_TOTAL = 64
UNITS = B_TOTAL * G


def _band_info():
    ys = (np.arange(G) + 0.5) * H / G
    seg = np.argmin(np.abs(np.arange(H)[:, None].astype(np.float32) - ys[None, :]), axis=1)
    out = []
    for k in range(G):
        rows = np.where(seg == k)[0]
        assert np.all(np.diff(rows) == 1)
        out.append((int(rows[0]), len(rows)))
    return out


def _build_idx_tab():
    bands = _band_info()
    tab = np.full((G, UNIT_COLS), SENTINEL, dtype=np.int32)
    yload_tab = np.zeros((G,), dtype=np.int32)
    for by, (y0, h) in enumerate(bands):
        yload = min(y0, H - 17)
        yload_tab[by] = yload
        roff = y0 - yload
        for bx, (x0, w) in enumerate(bands):
            n = h * w
            j = np.arange(n)
            off = (roff + j // w) * H + (x0 + j % w)
            for c in range(N_CH):
                base = bx * SEG_COLS + c * MAX_PIX
                tab[by, base:base + n] = c * BAND_W + off
    return tab, yload_tab


_IDX_TAB, _YLOAD_TAB = _build_idx_tab()
assert np.all(_YLOAD_TAB == np.minimum(np.where(np.arange(G) > 0, np.arange(G) * 16 + 1, 0), H - 17))


def _sc_gather(imgf, idx_tab, n_batch):
    info = plsc.get_sparse_core_info()
    nw = info.num_cores * info.num_subcores
    units = n_batch * G
    assert units % nw == 0
    per = units // nw
    mesh = plsc.VectorSubcoreMesh(core_axis_name="c", subcore_axis_name="s")

    @functools.partial(
        pl.kernel,
        mesh=mesh,
        compiler_params=pltpu.CompilerParams(needs_layout_passes=False),
        out_type=jax.ShapeDtypeStruct((n_batch * G * UNIT_COLS,), jnp.float32),
        scratch_types=[
            pltpu.VMEM((BAND_BUF,), jnp.float32),
            pltpu.VMEM((BAND_BUF,), jnp.float32),
            pltpu.VMEM((UNIT_COLS,), jnp.int32),
            pltpu.VMEM((UNIT_COLS,), jnp.int32),
            pltpu.VMEM((UNIT_COLS,), jnp.float32),
            pltpu.VMEM((UNIT_COLS,), jnp.float32),
            pltpu.SemaphoreType.DMA,
            pltpu.SemaphoreType.DMA,
            pltpu.SemaphoreType.DMA,
            pltpu.SemaphoreType.DMA,
        ],
    )
    def k(img_hbm, tab_hbm, out_hbm, band0_v, band1_v, idx0_v, idx1_v,
          buf0_v, buf1_v, sb0, sb1, so0, so1):
        wid = lax.axis_index("s") * info.num_cores + lax.axis_index("c")
        u0 = wid * per
        bands_v = (band0_v, band1_v)
        idxs_v = (idx0_v, idx1_v)
        bufs_v = (buf0_v, buf1_v)
        sbands = (sb0, sb1)
        souts = (so0, so1)
        for p in range(2):
            bands_v[p][pl.ds(SENTINEL, 16)] = jnp.zeros((16,), jnp.float32)

        def unit_scalars(i):
            u = u0 + i
            by = u // n_batch
            b = u - by * n_batch
            y0 = jnp.where(by > 0, by * 16 + 1, 0)
            yload = jnp.minimum(y0, H - 17)
            return u, by, b, yload

        def start_band(i, p):
            _, by, b, yload = unit_scalars(i)
            copies = []
            for c in range(N_CH):
                copies.append(pltpu.async_copy(
                    img_hbm.at[pl.ds((b * N_CH + c) * (H * H) + yload * H, BAND_W)],
                    bands_v[p].at[pl.ds(c * BAND_W, BAND_W)],
                    sbands[p],
                ))
            copies.append(pltpu.async_copy(
                tab_hbm.at[pl.ds(by * UNIT_COLS, UNIT_COLS)], idxs_v[p], sbands[p]))
            return copies

        pend_band = {0: start_band(0, 0)}
        pend_out = {}

        for i in range(per):
            p = i & 1
            u, by, b, yload = unit_scalars(i)
            for h in pend_band.pop(i):
                h.wait()
            if i + 1 < per:
                pend_band[i + 1] = start_band(i + 1, 1 - p)
            if i - 2 in pend_out:
                pend_out.pop(i - 2).wait()

            def gather_body(kk, _2):
                base = kk * 64
                for t in range(4):
                    ind = idxs_v[p][pl.ds(base + t * 16, 16)]
                    bufs_v[p][pl.ds(base + t * 16, 16)] = plsc.load_gather(
                        bands_v[p], [ind])
                return 0

            lax.fori_loop(0, UNIT_COLS // 64, gather_body, 0, unroll=False)
            for base in range((UNIT_COLS // 64) * 64, UNIT_COLS, 16):
                ind = idxs_v[p][pl.ds(base, 16)]
                bufs_v[p][pl.ds(base, 16)] = plsc.load_gather(bands_v[p], [ind])
            pend_out[i] = pltpu.async_copy(
                bufs_v[p], out_hbm.at[pl.ds((b * G + by) * UNIT_COLS, UNIT_COLS)],
                souts[p])

        for h in pend_out.values():
            h.wait()

    return k(imgf, idx_tab)


def _tc_matmul(feats, Wm, bias2, n_batch):
    BB = 4

    def body(f_ref, w_ref, b_ref, o_ref):
        o_ref[...] = (
            lax.dot_general(
                f_ref[...], w_ref[...],
                (((2,), (0,)), ((), ())),
                preferred_element_type=jnp.float32,
            )
            + b_ref[...][None]
        )

    return pl.pallas_call(
        body,
        grid=(n_batch // BB,),
        in_specs=[
            pl.BlockSpec((BB, N_SEG, SEG_COLS), lambda i: (i, 0, 0)),
            pl.BlockSpec((SEG_COLS, 128), lambda i: (0, 0)),
            pl.BlockSpec((1, 128), lambda i: (0, 0)),
        ],
        out_specs=pl.BlockSpec((BB, N_SEG, 128), lambda i: (i, 0, 0)),
        out_shape=jax.ShapeDtypeStruct((n_batch, N_SEG, 128), jnp.float32),
    )(feats, Wm, bias2)


def kernel(img, W, b):
    NCHUNK = 2
    bc = B_TOTAL // NCHUNK
    imgf = img.reshape(B_TOTAL * N_CH * H * H)
    tab = jnp.asarray(_IDX_TAB).reshape(-1)
    bias2 = b.reshape(1, 128)
    outs = []
    for h in range(NCHUNK):
        img_h = lax.dynamic_slice_in_dim(imgf, h * bc * N_CH * H * H,
                                         bc * N_CH * H * H)
        featsf = _sc_gather(img_h, tab, bc)
        feats = featsf.reshape(bc, N_SEG, SEG_COLS)
        outs.append(_tc_matmul(feats, W, bias2, bc))
    return jnp.concatenate(outs, axis=0)


# speedup vs baseline: 1.3816x; 1.0249x over previous
import functools

import numpy as np
import jax
import jax.numpy as jnp
from jax import lax
from jax.experimental import pallas as pl
from jax.experimental.pallas import tpu as pltpu
from jax.experimental.pallas import tpu_sc as plsc

H = 224
G = 14
N_SEG = G * G
MAX_PIX = 400
N_CH = 3
SEG_COLS = N_CH * MAX_PIX
BAND_W = 17 * H
SENTINEL = N_CH * BAND_W
BAND_BUF = SENTINEL + 16
UNIT_COLS = G * SEG_COLS
B_TOTAL = 64
UNITS = B_TOTAL * G


def _band_info():
    ys = (np.arange(G) + 0.5) * H / G
    seg = np.argmin(np.abs(np.arange(H)[:, None].astype(np.float32) - ys[None, :]), axis=1)
    out = []
    for k in range(G):
        rows = np.where(seg == k)[0]
        assert np.all(np.diff(rows) == 1)
        out.append((int(rows[0]), len(rows)))
    return out


def _build_idx_tab():
    bands = _band_info()
    tab = np.full((G, UNIT_COLS), SENTINEL, dtype=np.int32)
    yload_tab = np.zeros((G,), dtype=np.int32)
    for by, (y0, h) in enumerate(bands):
        yload = min(y0, H - 17)
        yload_tab[by] = yload
        roff = y0 - yload
        for bx, (x0, w) in enumerate(bands):
            n = h * w
            j = np.arange(n)
            off = (roff + j // w) * H + (x0 + j % w)
            for c in range(N_CH):
                base = bx * SEG_COLS + c * MAX_PIX
                tab[by, base:base + n] = c * BAND_W + off
    return tab, yload_tab


_IDX_TAB, _YLOAD_TAB = _build_idx_tab()
assert np.all(_YLOAD_TAB == np.minimum(np.where(np.arange(G) > 0, np.arange(G) * 16 + 1, 0), H - 17))


def _sc_gather(imgf, idx_tab, n_batch):
    info = plsc.get_sparse_core_info()
    nw = info.num_cores * info.num_subcores
    units = n_batch * G
    assert units % nw == 0
    per = units // nw
    mesh = plsc.VectorSubcoreMesh(core_axis_name="c", subcore_axis_name="s")

    @functools.partial(
        pl.kernel,
        mesh=mesh,
        compiler_params=pltpu.CompilerParams(needs_layout_passes=False),
        out_type=jax.ShapeDtypeStruct((n_batch * G * UNIT_COLS,), jnp.float32),
        scratch_types=[
            pltpu.VMEM((BAND_BUF,), jnp.float32),
            pltpu.VMEM((BAND_BUF,), jnp.float32),
            pltpu.VMEM((UNIT_COLS,), jnp.int32),
            pltpu.VMEM((UNIT_COLS,), jnp.int32),
            pltpu.VMEM((UNIT_COLS,), jnp.float32),
            pltpu.VMEM((UNIT_COLS,), jnp.float32),
            pltpu.SemaphoreType.DMA,
            pltpu.SemaphoreType.DMA,
            pltpu.SemaphoreType.DMA,
            pltpu.SemaphoreType.DMA,
        ],
    )
    def k(img_hbm, tab_hbm, out_hbm, band0_v, band1_v, idx0_v, idx1_v,
          buf0_v, buf1_v, sb0, sb1, so0, so1):
        wid = lax.axis_index("s") * info.num_cores + lax.axis_index("c")
        u0 = wid * per
        bands_v = (band0_v, band1_v)
        idxs_v = (idx0_v, idx1_v)
        bufs_v = (buf0_v, buf1_v)
        sbands = (sb0, sb1)
        souts = (so0, so1)
        for p in range(2):
            bands_v[p][pl.ds(SENTINEL, 16)] = jnp.zeros((16,), jnp.float32)

        def unit_scalars(i):
            u = u0 + i
            by = u // n_batch
            b = u - by * n_batch
            y0 = jnp.where(by > 0, by * 16 + 1, 0)
            yload = jnp.minimum(y0, H - 17)
            return u, by, b, yload

        def start_band(i, p):
            _, by, b, yload = unit_scalars(i)
            copies = []
            for c in range(N_CH):
                copies.append(pltpu.async_copy(
                    img_hbm.at[pl.ds((b * N_CH + c) * (H * H) + yload * H, BAND_W)],
                    bands_v[p].at[pl.ds(c * BAND_W, BAND_W)],
                    sbands[p],
                ))
            copies.append(pltpu.async_copy(
                tab_hbm.at[pl.ds(by * UNIT_COLS, UNIT_COLS)], idxs_v[p], sbands[p]))
            return copies

        pend_band = {0: start_band(0, 0)}
        pend_out = {}

        for i in range(per):
            p = i & 1
            u, by, b, yload = unit_scalars(i)
            for h in pend_band.pop(i):
                h.wait()
            if i + 1 < per:
                pend_band[i + 1] = start_band(i + 1, 1 - p)
            if i - 2 in pend_out:
                pend_out.pop(i - 2).wait()

            def gather_body(kk, _2):
                base = kk * 64
                for t in range(4):
                    ind = idxs_v[p][pl.ds(base + t * 16, 16)]
                    bufs_v[p][pl.ds(base + t * 16, 16)] = plsc.load_gather(
                        bands_v[p], [ind])
                return 0

            lax.fori_loop(0, UNIT_COLS // 64, gather_body, 0, unroll=False)
            for base in range((UNIT_COLS // 64) * 64, UNIT_COLS, 16):
                ind = idxs_v[p][pl.ds(base, 16)]
                bufs_v[p][pl.ds(base, 16)] = plsc.load_gather(bands_v[p], [ind])
            pend_out[i] = pltpu.async_copy(
                bufs_v[p], out_hbm.at[pl.ds((b * G + by) * UNIT_COLS, UNIT_COLS)],
                souts[p])

        for h in pend_out.values():
            h.wait()

    return k(imgf, idx_tab)


def _tc_matmul(feats, Wm, bias2, n_batch):
    BB = 4

    def body(f_ref, w_ref, b_ref, o_ref):
        o_ref[...] = (
            lax.dot_general(
                f_ref[...], w_ref[...],
                (((2,), (0,)), ((), ())),
                preferred_element_type=jnp.float32,
            )
            + b_ref[...][None]
        )

    return pl.pallas_call(
        body,
        grid=(n_batch // BB,),
        in_specs=[
            pl.BlockSpec((BB, N_SEG, SEG_COLS), lambda i: (i, 0, 0)),
            pl.BlockSpec((SEG_COLS, 128), lambda i: (0, 0)),
            pl.BlockSpec((1, 128), lambda i: (0, 0)),
        ],
        out_specs=pl.BlockSpec((BB, N_SEG, 128), lambda i: (i, 0, 0)),
        out_shape=jax.ShapeDtypeStruct((n_batch, N_SEG, 128), jnp.float32),
    )(feats, Wm, bias2)


def kernel(img, W, b):
    NCHUNK = 4
    bc = B_TOTAL // NCHUNK
    imgf = img.reshape(B_TOTAL * N_CH * H * H)
    tab = jnp.asarray(_IDX_TAB).reshape(-1)
    bias2 = b.reshape(1, 128)
    outs = []
    for h in range(NCHUNK):
        img_h = lax.dynamic_slice_in_dim(imgf, h * bc * N_CH * H * H,
                                         bc * N_CH * H * H)
        featsf = _sc_gather(img_h, tab, bc)
        feats = featsf.reshape(bc, N_SEG, SEG_COLS)
        outs.append(_tc_matmul(feats, W, bias2, bc))
    return jnp.concatenate(outs, axis=0)


# speedup vs baseline: 1.7806x; 1.2888x over previous
import functools

import numpy as np
import jax
import jax.numpy as jnp
from jax import lax
from jax.experimental import pallas as pl
from jax.experimental.pallas import tpu as pltpu
from jax.experimental.pallas import tpu_sc as plsc

H = 224
G = 14
N_SEG = G * G
MAX_PIX = 400
N_CH = 3
SEG_COLS = N_CH * MAX_PIX
BAND_W = 17 * H
SENTINEL = N_CH * BAND_W
BAND_BUF = SENTINEL + 16
UNIT_COLS = G * SEG_COLS
B_TOTAL = 64
UNITS = B_TOTAL * G


def _band_info():
    ys = (np.arange(G) + 0.5) * H / G
    seg = np.argmin(np.abs(np.arange(H)[:, None].astype(np.float32) - ys[None, :]), axis=1)
    out = []
    for k in range(G):
        rows = np.where(seg == k)[0]
        assert np.all(np.diff(rows) == 1)
        out.append((int(rows[0]), len(rows)))
    return out


WIN_ROWS = 24
BAND_ROWS = N_CH * WIN_ROWS + 1
SENT_ROW = N_CH * WIN_ROWS


def _build_idx_tab():
    bands = _band_info()
    tab = np.full((G, UNIT_COLS), (SENT_ROW << 10), dtype=np.int32)
    ys_tab = np.zeros((G,), dtype=np.int32)
    for by, (y0, h) in enumerate(bands):
        ys = min(y0 - y0 % 8, H - WIN_ROWS)
        ys_tab[by] = ys
        roff = y0 - ys
        assert 0 <= roff and roff + h <= WIN_ROWS
        for bx, (x0, w) in enumerate(bands):
            n = h * w
            j = np.arange(n)
            iy = roff + j // w
            ix = x0 + j % w
            for c in range(N_CH):
                base = bx * SEG_COLS + c * MAX_PIX
                tab[by, base:base + n] = ((c * WIN_ROWS + iy) << 10) | ix
    return tab, ys_tab


_IDX_TAB, _YS_TAB = _build_idx_tab()
assert np.all(_YS_TAB == np.minimum(np.arange(G) * 16, H - WIN_ROWS))


def _sc_gather(img, idx_tab, n_batch, b_off):
    info = plsc.get_sparse_core_info()
    nw = info.num_cores * info.num_subcores
    units = n_batch * G
    assert units % nw == 0
    per = units // nw
    mesh = plsc.VectorSubcoreMesh(core_axis_name="c", subcore_axis_name="s")

    @functools.partial(
        pl.kernel,
        mesh=mesh,
        compiler_params=pltpu.CompilerParams(needs_layout_passes=False),
        out_type=jax.ShapeDtypeStruct((n_batch * G * UNIT_COLS,), jnp.float32),
        scratch_types=[
            pltpu.VMEM((BAND_ROWS, H), jnp.float32),
            pltpu.VMEM((BAND_ROWS, H), jnp.float32),
            pltpu.VMEM((UNIT_COLS,), jnp.int32),
            pltpu.VMEM((UNIT_COLS,), jnp.int32),
            pltpu.VMEM((UNIT_COLS,), jnp.float32),
            pltpu.VMEM((UNIT_COLS,), jnp.float32),
            pltpu.SemaphoreType.DMA,
            pltpu.SemaphoreType.DMA,
            pltpu.SemaphoreType.DMA,
            pltpu.SemaphoreType.DMA,
        ],
    )
    def k(img_hbm, tab_hbm, out_hbm, band0_v, band1_v, idx0_v, idx1_v,
          buf0_v, buf1_v, sb0, sb1, so0, so1):
        wid = lax.axis_index("s") * info.num_cores + lax.axis_index("c")
        u0 = wid * per
        bands_v = (band0_v, band1_v)
        idxs_v = (idx0_v, idx1_v)
        bufs_v = (buf0_v, buf1_v)
        sbands = (sb0, sb1)
        souts = (so0, so1)
        for p in range(2):
            bands_v[p][SENT_ROW, pl.ds(0, 16)] = jnp.zeros((16,), jnp.float32)

        def unit_scalars(i):
            u = u0 + i
            by = u // n_batch
            b = u - by * n_batch
            ys = jnp.minimum(by * 16, H - WIN_ROWS)
            return u, by, b, ys

        def start_band(i, p):
            _, by, b, ys = unit_scalars(i)
            copies = []
            for c in range(N_CH):
                copies.append(pltpu.async_copy(
                    img_hbm.at[b_off + b, c, pl.ds(ys, WIN_ROWS), :],
                    bands_v[p].at[pl.ds(c * WIN_ROWS, WIN_ROWS), :],
                    sbands[p],
                ))
            copies.append(pltpu.async_copy(
                tab_hbm.at[pl.ds(by * UNIT_COLS, UNIT_COLS)], idxs_v[p], sbands[p]))
            return copies

        pend_band = {0: start_band(0, 0)}
        pend_out = {}

        for i in range(per):
            p = i & 1
            u, by, b, ys = unit_scalars(i)
            for h in pend_band.pop(i):
                h.wait()
            if i + 1 < per:
                pend_band[i + 1] = start_band(i + 1, 1 - p)
            if i - 2 in pend_out:
                pend_out.pop(i - 2).wait()

            def gather_body(kk, _2):
                base = kk * 64
                for t in range(4):
                    pk = idxs_v[p][pl.ds(base + t * 16, 16)]
                    iy = lax.shift_right_logical(pk, 10)
                    ix = lax.bitwise_and(pk, 1023)
                    bufs_v[p][pl.ds(base + t * 16, 16)] = plsc.load_gather(
                        bands_v[p], [iy, ix])
                return 0

            lax.fori_loop(0, UNIT_COLS // 64, gather_body, 0, unroll=False)
            for base in range((UNIT_COLS // 64) * 64, UNIT_COLS, 16):
                pk = idxs_v[p][pl.ds(base, 16)]
                iy = lax.shift_right_logical(pk, 10)
                ix = lax.bitwise_and(pk, 1023)
                bufs_v[p][pl.ds(base, 16)] = plsc.load_gather(bands_v[p], [iy, ix])
            pend_out[i] = pltpu.async_copy(
                bufs_v[p], out_hbm.at[pl.ds((b * G + by) * UNIT_COLS, UNIT_COLS)],
                souts[p])

        for h in pend_out.values():
            h.wait()

    return k(img, idx_tab)


def _tc_matmul(feats, Wm, bias2, n_batch):
    BB = 4

    def body(f_ref, w_ref, b_ref, o_ref):
        o_ref[...] = (
            lax.dot_general(
                f_ref[...], w_ref[...],
                (((2,), (0,)), ((), ())),
                preferred_element_type=jnp.float32,
            )
            + b_ref[...][None]
        )

    return pl.pallas_call(
        body,
        grid=(n_batch // BB,),
        in_specs=[
            pl.BlockSpec((BB, N_SEG, SEG_COLS), lambda i: (i, 0, 0)),
            pl.BlockSpec((SEG_COLS, 128), lambda i: (0, 0)),
            pl.BlockSpec((1, 128), lambda i: (0, 0)),
        ],
        out_specs=pl.BlockSpec((BB, N_SEG, 128), lambda i: (i, 0, 0)),
        out_shape=jax.ShapeDtypeStruct((n_batch, N_SEG, 128), jnp.float32),
    )(feats, Wm, bias2)


def kernel(img, W, b):
    NCHUNK = 4
    bc = B_TOTAL // NCHUNK
    tab = jnp.asarray(_IDX_TAB).reshape(-1)
    bias2 = b.reshape(1, 128)
    outs = []
    for h in range(NCHUNK):
        featsf = _sc_gather(img, tab, bc, h * bc)
        feats = featsf.reshape(bc, N_SEG, SEG_COLS)
        outs.append(_tc_matmul(feats, W, bias2, bc))
    return jnp.concatenate(outs, axis=0)


# speedup vs baseline: 1.7996x; 1.0107x over previous
import functools

import numpy as np
import jax
import jax.numpy as jnp
from jax import lax
from jax.experimental import pallas as pl
from jax.experimental.pallas import tpu as pltpu
from jax.experimental.pallas import tpu_sc as plsc

H = 224
G = 14
N_SEG = G * G
MAX_PIX = 400
N_CH = 3
SEG_COLS = N_CH * MAX_PIX
BAND_W = 17 * H
SENTINEL = N_CH * BAND_W
BAND_BUF = SENTINEL + 16
UNIT_COLS = G * SEG_COLS
B_TOTAL = 64
UNITS = B_TOTAL * G


def _band_info():
    ys = (np.arange(G) + 0.5) * H / G
    seg = np.argmin(np.abs(np.arange(H)[:, None].astype(np.float32) - ys[None, :]), axis=1)
    out = []
    for k in range(G):
        rows = np.where(seg == k)[0]
        assert np.all(np.diff(rows) == 1)
        out.append((int(rows[0]), len(rows)))
    return out


WIN_ROWS = 24
BAND_ROWS = N_CH * WIN_ROWS + 1
SENT_ROW = N_CH * WIN_ROWS
SEG_COLS_P = 1280
PAD_ROWS = 16
UNIT_POS = PAD_ROWS * SEG_COLS_P
BATCH_WORDS = G * UNIT_POS


def _build_idx_tab():
    bands = _band_info()
    sent = SENT_ROW << 10
    tab = np.full((G, UNIT_POS), sent, dtype=np.int32)
    ys_tab = np.zeros((G,), dtype=np.int32)
    pos = np.arange(UNIT_POS)
    piece, rem = pos // (UNIT_POS // 2), pos % (UNIT_POS // 2)
    cb, rr, cc = rem // 1024, (rem % 1024) // 128, pos % 128
    lr = piece * 8 + rr
    col = cb * 128 + cc
    for by, (y0, h) in enumerate(bands):
        ys = min(y0 - y0 % 8, H - WIN_ROWS)
        ys_tab[by] = ys
        roff = y0 - ys
        assert 0 <= roff and roff + h <= WIN_ROWS
        row = np.full((UNIT_POS,), sent, dtype=np.int32)
        for bx, (x0, w) in enumerate(bands):
            n = h * w
            sel = (lr == bx) & (col < SEG_COLS)
            c = col[sel] // MAX_PIX
            j = col[sel] % MAX_PIX
            iy = roff + j // w
            ix = x0 + j % w
            v = np.where(j < n, ((c * WIN_ROWS + iy) << 10) | ix, sent)
            row[sel] = v
        tab[by] = row
    return tab, ys_tab


_IDX_TAB, _YS_TAB = _build_idx_tab()
assert np.all(_YS_TAB == np.minimum(np.arange(G) * 16, H - WIN_ROWS))


def _sc_gather(img, idx_tab, n_batch, b_off):
    info = plsc.get_sparse_core_info()
    nw = info.num_cores * info.num_subcores
    units = n_batch * G
    assert units % nw == 0
    per = units // nw
    mesh = plsc.VectorSubcoreMesh(core_axis_name="c", subcore_axis_name="s")

    @functools.partial(
        pl.kernel,
        mesh=mesh,
        compiler_params=pltpu.CompilerParams(needs_layout_passes=False),
        out_type=jax.ShapeDtypeStruct((n_batch * BATCH_WORDS,), jnp.float32),
        scratch_types=[
            pltpu.VMEM((BAND_ROWS, H), jnp.float32),
            pltpu.VMEM((BAND_ROWS, H), jnp.float32),
            pltpu.VMEM((UNIT_POS,), jnp.int32),
            pltpu.VMEM((UNIT_POS,), jnp.int32),
            pltpu.VMEM((UNIT_POS,), jnp.float32),
            pltpu.VMEM((UNIT_POS,), jnp.float32),
            pltpu.SemaphoreType.DMA,
            pltpu.SemaphoreType.DMA,
            pltpu.SemaphoreType.DMA,
            pltpu.SemaphoreType.DMA,
        ],
    )
    def k(img_hbm, tab_hbm, out_hbm, band0_v, band1_v, idx0_v, idx1_v,
          buf0_v, buf1_v, sb0, sb1, so0, so1):
        wid = lax.axis_index("s") * info.num_cores + lax.axis_index("c")
        u0 = wid * per
        bands_v = (band0_v, band1_v)
        idxs_v = (idx0_v, idx1_v)
        bufs_v = (buf0_v, buf1_v)
        sbands = (sb0, sb1)
        souts = (so0, so1)
        for p in range(2):
            bands_v[p][SENT_ROW, pl.ds(0, 16)] = jnp.zeros((16,), jnp.float32)

        def unit_scalars(i):
            u = u0 + i
            by = u // n_batch
            b = u - by * n_batch
            ys = jnp.minimum(by * 16, H - WIN_ROWS)
            return u, by, b, ys

        def start_band(i, p):
            _, by, b, ys = unit_scalars(i)
            copies = []
            for c in range(N_CH):
                copies.append(pltpu.async_copy(
                    img_hbm.at[b_off + b, c, pl.ds(ys, WIN_ROWS), :],
                    bands_v[p].at[pl.ds(c * WIN_ROWS, WIN_ROWS), :],
                    sbands[p],
                ))
            copies.append(pltpu.async_copy(
                tab_hbm.at[pl.ds(by * UNIT_POS, UNIT_POS)], idxs_v[p], sbands[p]))
            return copies

        pend_band = {0: start_band(0, 0)}
        pend_out = {}

        for i in range(per):
            p = i & 1
            u, by, b, ys = unit_scalars(i)
            for h in pend_band.pop(i):
                h.wait()
            if i + 1 < per:
                pend_band[i + 1] = start_band(i + 1, 1 - p)
            if i - 2 in pend_out:
                pend_out.pop(i - 2).wait()

            def gather_body(kk, _2):
                base = kk * 64
                for t in range(4):
                    pk = idxs_v[p][pl.ds(base + t * 16, 16)]
                    iy = lax.shift_right_logical(pk, 10)
                    ix = lax.bitwise_and(pk, 1023)
                    bufs_v[p][pl.ds(base + t * 16, 16)] = plsc.load_gather(
                        bands_v[p], [iy, ix])
                return 0

            lax.fori_loop(0, UNIT_POS // 64, gather_body, 0, unroll=False)
            pend_out[i] = pltpu.async_copy(
                bufs_v[p],
                out_hbm.at[pl.ds(b * BATCH_WORDS + by * UNIT_POS, UNIT_POS)],
                souts[p])

        for h in pend_out.values():
            h.wait()

    return k(img, idx_tab)


def _tc_matmul(feats4, W3, bias2, n_batch):
    BB = 4
    TBB = BB * G * 2
    RWS = TBB * 8

    def body(f_ref, w_ref, b_ref, o_ref):
        acc = lax.dot(
            f_ref[:, 0, :, :].reshape(RWS, 128), w_ref[0],
            preferred_element_type=jnp.float32)
        for cb in range(1, 10):
            acc = acc + lax.dot(
                f_ref[:, cb, :, :].reshape(RWS, 128), w_ref[cb],
                preferred_element_type=jnp.float32)
        acc = acc + b_ref[...]
        o_ref[...] = acc.reshape(BB, G, PAD_ROWS, 128)

    return pl.pallas_call(
        body,
        grid=(n_batch // BB,),
        in_specs=[
            pl.BlockSpec((TBB, 10, 8, 128), lambda i: (i, 0, 0, 0)),
            pl.BlockSpec((10, 128, 128), lambda i: (0, 0, 0)),
            pl.BlockSpec((1, 128), lambda i: (0, 0)),
        ],
        out_specs=pl.BlockSpec((BB, G, PAD_ROWS, 128), lambda i: (i, 0, 0, 0)),
        out_shape=jax.ShapeDtypeStruct((n_batch, G, PAD_ROWS, 128), jnp.float32),
    )(feats4, W3, bias2)


def kernel(img, W, b):
    NCHUNK = 4
    bc = B_TOTAL // NCHUNK
    tab = jnp.asarray(_IDX_TAB).reshape(-1)
    bias2 = b.reshape(1, 128)
    W3 = jnp.concatenate(
        [W, jnp.zeros((SEG_COLS_P - SEG_COLS, 128), W.dtype)],
        axis=0).reshape(10, 128, 128)
    outs = []
    for h in range(NCHUNK):
        featsf = _sc_gather(img, tab, bc, h * bc)
        feats4 = featsf.reshape(bc * G * 2, 10, 8, 128)
        outs.append(_tc_matmul(feats4, W3, bias2, bc))
    outp = jnp.concatenate(outs, axis=0)
    return outp[:, :, :G, :].reshape(B_TOTAL, N_SEG, 128)
